# baseline (device time: 294598 ns/iter reference)
import jax
import jax.numpy as jnp
from jax import lax
from jax.experimental import pallas as pl
from jax.experimental.pallas import tpu as pltpu

N_DEV = 32
B = 2
SQ = 512
SKV = 512
H_PER = 8
DH = 64
D_MODEL = 768
ROWS = B * SQ
CH = ROWS // N_DEV


def _grp(a):
    return a.reshape(2, 4, 64, H_PER, DH).transpose(3, 1, 0, 2, 4).reshape(
        H_PER * 4, 128, DH)


def _attn_body(m_ref, x_ref, wq_ref, k_ref, v_ref, wo_ref, out_ref):
    del m_ref
    x2 = x_ref[0]
    q = jnp.dot(x2, wq_ref[...], preferred_element_type=jnp.float32)
    qg = _grp(q.reshape(SQ, H_PER, DH))
    kg = _grp(k_ref[0])
    vg = _grp(v_ref[0])
    scores = lax.dot_general(
        qg, kg, (((2,), (2,)), ((0,), (0,))),
        preferred_element_type=jnp.float32) * 0.125
    m = jnp.max(scores, axis=-1, keepdims=True)
    w = jnp.exp(scores - m)
    w = w / jnp.sum(w, axis=-1, keepdims=True)
    ctx = lax.dot_general(
        w, vg, (((2,), (1,)), ((0,), (0,))),
        preferred_element_type=jnp.float32)
    ctx2 = ctx.reshape(H_PER, 4, 2, 64, DH).transpose(2, 1, 3, 0, 4).reshape(
        SQ, H_PER * DH)
    out_ref[0] = jnp.dot(ctx2, wo_ref[...], preferred_element_type=jnp.float32)


def _ar_body(p_ref, out_ref, acc_ref, send1, recv1, send2, recv2):
    my = lax.axis_index("i")

    p1 = []
    for p in range(N_DEV):
        r = pltpu.make_async_remote_copy(
            src_ref=p_ref.at[pl.ds(p * CH, CH)],
            dst_ref=acc_ref.at[pl.ds(my * CH, CH)],
            send_sem=send1.at[p],
            recv_sem=recv1.at[my],
            device_id=(p,),
            device_id_type=pl.DeviceIdType.MESH,
        )
        p1.append(r)

        @pl.when(my != p)
        def _(r=r):
            r.start()

    for s in range(N_DEV):
        rcv = pltpu.make_async_remote_copy(
            src_ref=acc_ref.at[pl.ds(s * CH, CH)],
            dst_ref=acc_ref.at[pl.ds(s * CH, CH)],
            send_sem=send1.at[s],
            recv_sem=recv1.at[s],
            device_id=(s,),
            device_id_type=pl.DeviceIdType.MESH,
        )

        @pl.when(my != s)
        def _(rcv=rcv):
            rcv.wait_recv()

    own = p_ref[pl.ds(my * CH, CH), :]
    acc = acc_ref[...].reshape(N_DEV, CH, D_MODEL)
    sidx = lax.broadcasted_iota(jnp.int32, (N_DEV, 1, 1), 0)
    red = own + jnp.sum(jnp.where(sidx == my, 0.0, acc), axis=0)
    out_ref[pl.ds(my * CH, CH), :] = red

    p2 = []
    for p in range(N_DEV):
        r = pltpu.make_async_remote_copy(
            src_ref=out_ref.at[pl.ds(my * CH, CH)],
            dst_ref=out_ref.at[pl.ds(my * CH, CH)],
            send_sem=send2.at[p],
            recv_sem=recv2.at[my],
            device_id=(p,),
            device_id_type=pl.DeviceIdType.MESH,
        )
        p2.append(r)

        @pl.when(my != p)
        def _(r=r):
            r.start()

    for s in range(N_DEV):
        rcv = pltpu.make_async_remote_copy(
            src_ref=out_ref.at[pl.ds(s * CH, CH)],
            dst_ref=out_ref.at[pl.ds(s * CH, CH)],
            send_sem=send2.at[s],
            recv_sem=recv2.at[s],
            device_id=(s,),
            device_id_type=pl.DeviceIdType.MESH,
        )

        @pl.when(my != s)
        def _(rcv=rcv):
            rcv.wait_recv()

    for p in range(N_DEV):
        @pl.when(my != p)
        def _(r=p1[p]):
            r.wait_send()

        @pl.when(my != p)
        def _(r=p2[p]):
            r.wait_send()


def kernel(x, Wq, K_ext, V_ext, Wo):
    my = lax.axis_index("i")
    midx = jnp.reshape(my, (1,)).astype(jnp.int32)

    HD = H_PER * DH
    partial = pl.pallas_call(
        _attn_body,
        grid_spec=pltpu.PrefetchScalarGridSpec(
            num_scalar_prefetch=1,
            grid=(B,),
            in_specs=[
                pl.BlockSpec((1, SQ, D_MODEL), lambda b, m: (b, 0, 0)),
                pl.BlockSpec((D_MODEL, HD), lambda b, m: (0, 0)),
                pl.BlockSpec((1, SKV, H_PER, DH), lambda b, m: (b, 0, m[0], 0)),
                pl.BlockSpec((1, SKV, H_PER, DH), lambda b, m: (b, 0, m[0], 0)),
                pl.BlockSpec((HD, D_MODEL), lambda b, m: (0, 0)),
            ],
            out_specs=pl.BlockSpec((1, SQ, D_MODEL), lambda b, m: (b, 0, 0)),
        ),
        out_shape=jax.ShapeDtypeStruct((B, SQ, D_MODEL), jnp.float32),
    )(midx, x, Wq, K_ext, V_ext, Wo)

    out = pl.pallas_call(
        _ar_body,
        out_shape=jax.ShapeDtypeStruct((ROWS, D_MODEL), jnp.float32),
        in_specs=[pl.BlockSpec(memory_space=pltpu.VMEM)],
        out_specs=pl.BlockSpec(memory_space=pltpu.VMEM),
        scratch_shapes=[
            pltpu.VMEM((ROWS, D_MODEL), jnp.float32),
            pltpu.SemaphoreType.DMA((N_DEV,)),
            pltpu.SemaphoreType.DMA((N_DEV,)),
            pltpu.SemaphoreType.DMA((N_DEV,)),
            pltpu.SemaphoreType.DMA((N_DEV,)),
        ],
    )(partial.reshape(ROWS, D_MODEL))
    return out.reshape(B, SQ, D_MODEL)
